# baseline (speedup 1.0000x reference)
import functools

import jax
import jax.numpy as jnp
from jax import lax
from jax.experimental import pallas as pl
from jax.experimental.pallas import tpu as pltpu
from jax.experimental.pallas import tpu_sc as plsc

N = 10000
E = 320000
EL = 20000
C = 128
NC, NS, L = 2, 16, 16
NW = NC * NS
EPW = E // NW
K = 50
NCHUNK = EPW // K
SB = 8
NP = 10240
HR = NP // L
RPT = NP // NS
K2 = 80
NCH2 = EL // K2

_mesh = functools.partial(
    plsc.VectorSubcoreMesh, core_axis_name="c", subcore_axis_name="s",
    num_cores=NC, num_subcores=NS)


def _worker_id():
  return lax.axis_index("s") * NC + lax.axis_index("c")


def _deg_body(dst_hbm, degp_hbm, idx_v, hist_v, buf_v, res_v, shared_v):
  cid = lax.axis_index("c")
  sid = lax.axis_index("s")
  wid = _worker_id()
  CW = NP // NS

  @pl.loop(0, NP // L)
  def _zero(i):
    hist_v[pl.ds(i * L, L)] = jnp.zeros((L,), jnp.float32)

  pltpu.sync_copy(dst_hbm.at[pl.ds(wid * EPW, EPW)], idx_v)

  @pl.loop(0, EPW // L)
  def _acc(i):
    idx = idx_v[pl.ds(i * L, L)]
    plsc.addupdate_scatter(hist_v, [idx], jnp.ones((L,), jnp.float32))

  pltpu.sync_copy(hist_v, shared_v.at[sid])
  plsc.subcore_barrier()
  for r in range(NS):
    pltpu.sync_copy(shared_v.at[r, pl.ds(sid * CW, CW)], buf_v.at[r])

  @pl.loop(0, CW // L)
  def _sum(i):
    acc = jnp.zeros((L,), jnp.float32)
    for r in range(NS):
      acc += buf_v[r, pl.ds(i * L, L)]
    res_v[pl.ds(i * L, L)] = acc

  pltpu.sync_copy(res_v, degp_hbm.at[cid, pl.ds(sid * CW, CW)])


def _sc_degree(dst_flat):
  return pl.kernel(
      _deg_body,
      out_type=jax.ShapeDtypeStruct((NC, NP), jnp.float32),
      mesh=_mesh(),
      compiler_params=pltpu.CompilerParams(needs_layout_passes=False),
      scratch_types=[
          pltpu.VMEM((EPW,), jnp.int32),
          pltpu.VMEM((NP,), jnp.float32),
          pltpu.VMEM((NS, NP // NS), jnp.float32),
          pltpu.VMEM((NP // NS,), jnp.float32),
          pltpu.VMEM_SHARED((NS, NP), jnp.float32),
      ],
  )(dst_flat)


def _edge_body(y_hbm, src_hbm, dst_hbm, z0_hbm, accp_hbm,
               src_v, dst_v, rows_v, acc_sh, sg0, sg1, ss0, ss1):
  cid = lax.axis_index("c")
  sid = lax.axis_index("s")
  wid = _worker_id()

  pltpu.sync_copy(z0_hbm.at[pl.ds(sid * RPT, RPT)],
                  acc_sh.at[pl.ds(sid * RPT, RPT)])
  plsc.subcore_barrier()

  def stage(ob):
    slot = lax.rem(ob, 2)
    pltpu.sync_copy(src_hbm.at[wid, pl.ds(ob * SB, SB)], src_v.at[slot])
    pltpu.sync_copy(dst_hbm.at[wid, pl.ds(ob * SB, SB)], dst_v.at[slot])

  def g_desc(j, buf, sem):
    return pltpu.make_async_copy(
        y_hbm.at[src_v.at[lax.rem(j // SB, 2), lax.rem(j, SB)]],
        rows_v.at[buf], sem)

  def s_start(j, buf, sem):
    pltpu.async_copy(
        rows_v.at[buf],
        acc_sh.at[dst_v.at[lax.rem(j // SB, 2), lax.rem(j, SB)]],
        sem, add=True)

  def s_wait(buf, sem):
    pltpu.make_async_copy(rows_v.at[buf], acc_sh.at[pl.ds(0, K)], sem).wait()

  stage(0)
  g_desc(0, 0, sg0).start()
  g_desc(1, 1, sg1).start()

  NP2 = NCHUNK // 2

  @pl.loop(0, NP2)
  def _pair(p):
    j0 = 2 * p
    j1 = j0 + 1
    g_desc(j0, 0, sg0).wait()
    s_start(j0, 0, ss0)
    g_desc(j1, 1, sg1).wait()
    s_start(j1, 1, ss1)

    @pl.when(p + 1 < NP2)
    def _next():
      @pl.when(lax.rem(j0 + 2, SB) == 0)
      def _stage():
        stage((j0 + 2) // SB)

      s_wait(0, ss0)
      g_desc(j0 + 2, 0, sg0).start()
      s_wait(1, ss1)
      g_desc(j1 + 2, 1, sg1).start()

  s_wait(0, ss0)
  s_wait(1, ss1)

  plsc.subcore_barrier()
  pltpu.sync_copy(acc_sh.at[pl.ds(sid * RPT, RPT)],
                  accp_hbm.at[cid, pl.ds(sid * RPT, RPT)])


def _sc_edge(y, src3, dst3, z0):
  return pl.kernel(
      _edge_body,
      out_type=jax.ShapeDtypeStruct((NC, NP, C), jnp.float32),
      mesh=_mesh(),
      compiler_params=pltpu.CompilerParams(needs_layout_passes=False),
      scratch_types=[
          pltpu.VMEM((2, SB, K), jnp.int32),
          pltpu.VMEM((2, SB, K), jnp.int32),
          pltpu.VMEM((2, K, C), jnp.float32),
          pltpu.VMEM_SHARED((NP, C), jnp.float32),
          pltpu.SemaphoreType.DMA,
          pltpu.SemaphoreType.DMA,
          pltpu.SemaphoreType.DMA,
          pltpu.SemaphoreType.DMA,
      ],
  )(y, src3, dst3, z0)


def _decode_body(z_hbm, la_hbm, lb_hbm, scores_hbm,
                 ia_v, ib_v, ra_v, rb_v, s_v, sema, semb):
  wid = _worker_id()

  @pl.loop(0, (NCH2 + NW - 1) // NW)
  def _chunk(i):
    c = wid + i * NW

    @pl.when(c < NCH2)
    def _do():
      pltpu.sync_copy(la_hbm.at[c], ia_v)
      pltpu.sync_copy(lb_hbm.at[c], ib_v)
      ca = pltpu.async_copy(z_hbm.at[ia_v], ra_v, sema)
      cb = pltpu.async_copy(z_hbm.at[ib_v], rb_v, semb)
      ca.wait()
      cb.wait()

      @pl.loop(0, K2 // L)
      def _blk(jb):
        rows = jnp.arange(L, dtype=jnp.int32) + jb * L

        def _chan(l, acc):
          cols = jnp.full((L,), l, jnp.int32)
          va = plsc.load_gather(ra_v, [rows, cols])
          vb = plsc.load_gather(rb_v, [rows, cols])
          return acc + va * vb

        s_v[pl.ds(jb * L, L)] = lax.fori_loop(
            0, C, _chan, jnp.zeros((L,), jnp.float32), unroll=8)

      pltpu.sync_copy(s_v, scores_hbm.at[pl.ds(c * K2, K2)])


def _sc_decode(z, la2, lb2):
  return pl.kernel(
      _decode_body,
      out_type=jax.ShapeDtypeStruct((EL,), jnp.float32),
      mesh=_mesh(),
      compiler_params=pltpu.CompilerParams(needs_layout_passes=False),
      scratch_types=[
          pltpu.VMEM((K2,), jnp.int32),
          pltpu.VMEM((K2,), jnp.int32),
          pltpu.VMEM((K2, C), jnp.float32),
          pltpu.VMEM((K2, C), jnp.float32),
          pltpu.VMEM((K2,), jnp.float32),
          pltpu.SemaphoreType.DMA,
          pltpu.SemaphoreType.DMA,
      ],
  )(z, la2, lb2)


B = 1000


def _dinv(degp_blk):
  deg = degp_blk[:, 0] + degp_blk[:, 1] + 1.0
  return lax.rsqrt(deg)


def _tc1_body(x_ref, w_ref, degp_ref, y_ref):
  dinv = _dinv(degp_ref[...])
  xw = jnp.dot(x_ref[...], w_ref[...], preferred_element_type=jnp.float32,
               precision=lax.Precision.HIGHEST)
  y_ref[...] = xw * dinv[:, None]


def _tc1(x, W1, degp):
  return pl.pallas_call(
      _tc1_body,
      grid=(N // B,),
      in_specs=[
          pl.BlockSpec((B, C), lambda i: (i, 0)),
          pl.BlockSpec((C, C), lambda i: (0, 0)),
          pl.BlockSpec((B, 2), lambda i: (i, 0)),
      ],
      out_specs=pl.BlockSpec((B, C), lambda i: (i, 0)),
      out_shape=jax.ShapeDtypeStruct((N, C), jnp.float32),
  )(x, W1, degp)


def _tc2_body(accp_ref, y1_ref, degp_ref, b1_ref, w_ref, y2_ref):
  dinv = _dinv(degp_ref[...])
  pre = (accp_ref[0] + accp_ref[1] + y1_ref[...]) * dinv[:, None] + b1_ref[0, :]
  h = jnp.maximum(pre, 0.0)
  xw = jnp.dot(h, w_ref[...], preferred_element_type=jnp.float32,
               precision=lax.Precision.HIGHEST)
  y2_ref[...] = xw * dinv[:, None]


def _tc2(accp, y1, degp, b1, W2):
  return pl.pallas_call(
      _tc2_body,
      grid=(N // B,),
      in_specs=[
          pl.BlockSpec((2, B, C), lambda i: (0, i, 0)),
          pl.BlockSpec((B, C), lambda i: (i, 0)),
          pl.BlockSpec((B, 2), lambda i: (i, 0)),
          pl.BlockSpec((1, C), lambda i: (0, 0)),
          pl.BlockSpec((C, C), lambda i: (0, 0)),
      ],
      out_specs=pl.BlockSpec((B, C), lambda i: (i, 0)),
      out_shape=jax.ShapeDtypeStruct((N, C), jnp.float32),
  )(accp, y1, degp, b1, W2)


def _tc3_body(accp_ref, y2_ref, degp_ref, b2_ref, z_ref):
  dinv = _dinv(degp_ref[...])
  z_ref[...] = ((accp_ref[0] + accp_ref[1] + y2_ref[...]) * dinv[:, None]
                + b2_ref[0, :])


def _tc3(accp, y2, degp, b2):
  return pl.pallas_call(
      _tc3_body,
      grid=(N // B,),
      in_specs=[
          pl.BlockSpec((2, B, C), lambda i: (0, i, 0)),
          pl.BlockSpec((B, C), lambda i: (i, 0)),
          pl.BlockSpec((B, 2), lambda i: (i, 0)),
          pl.BlockSpec((1, C), lambda i: (0, 0)),
      ],
      out_specs=pl.BlockSpec((B, C), lambda i: (i, 0)),
      out_shape=jax.ShapeDtypeStruct((N, C), jnp.float32),
  )(accp, y2, degp, b2)


def kernel(x, edge_index, edge_label_index, W1, b1, W2, b2):
  ei = edge_index.astype(jnp.int32)
  el = edge_label_index.astype(jnp.int32)
  src3 = ei[0].reshape(NW, NCHUNK, K)
  dst3 = ei[1].reshape(NW, NCHUNK, K)
  dst_flat = ei[1]
  la2 = el[0].reshape(NCH2, K2)
  lb2 = el[1].reshape(NCH2, K2)
  b1r = b1.reshape(1, C)
  b2r = b2.reshape(1, C)

  degp = _sc_degree(dst_flat)[:, :N].T
  y1 = _tc1(x, W1, degp)
  z0 = jnp.zeros((NP, C), jnp.float32)
  accp1 = _sc_edge(y1, src3 * 0, dst3, z0)[:, :N]
  y2 = _tc2(accp1, y1, degp, b1r, W2)
  accp2 = _sc_edge(y2, src3 * 0, dst3, z0)[:, :N]
  z = _tc3(accp2, y2, degp, b2r)
  scores = _sc_decode(z, la2, lb2)
  return scores


# speedup vs baseline: 43.7466x; 43.7466x over previous
import functools

import jax
import jax.numpy as jnp
from jax import lax
from jax.experimental import pallas as pl
from jax.experimental.pallas import tpu as pltpu
from jax.experimental.pallas import tpu_sc as plsc

N = 10000
E = 320000
EL = 20000
C = 128
NC, NS, L = 2, 16, 16
NW = NC * NS
EPW = E // NW
K = 25
NCHUNK = EPW // K
SB = 8
RING = 8
NP = 10240
HR = NP // L
RPT = NP // NS
K2 = 80
NCH2 = EL // K2

_mesh = functools.partial(
    plsc.VectorSubcoreMesh, core_axis_name="c", subcore_axis_name="s",
    num_cores=NC, num_subcores=NS)


def _worker_id():
  return lax.axis_index("s") * NC + lax.axis_index("c")


def _deg_body(dst_hbm, degp_hbm, idx_v, hist_v, buf_v, res_v, shared_v):
  cid = lax.axis_index("c")
  sid = lax.axis_index("s")
  wid = _worker_id()
  CW = NP // NS

  @pl.loop(0, NP // L)
  def _zero(i):
    hist_v[pl.ds(i * L, L)] = jnp.zeros((L,), jnp.float32)

  pltpu.sync_copy(dst_hbm.at[pl.ds(wid * EPW, EPW)], idx_v)

  @pl.loop(0, EPW // L)
  def _acc(i):
    idx = idx_v[pl.ds(i * L, L)]
    plsc.addupdate_scatter(hist_v, [idx], jnp.ones((L,), jnp.float32))

  pltpu.sync_copy(hist_v, shared_v.at[sid])
  plsc.subcore_barrier()
  for r in range(NS):
    pltpu.sync_copy(shared_v.at[r, pl.ds(sid * CW, CW)], buf_v.at[r])

  @pl.loop(0, CW // L)
  def _sum(i):
    acc = jnp.zeros((L,), jnp.float32)
    for r in range(NS):
      acc += buf_v[r, pl.ds(i * L, L)]
    res_v[pl.ds(i * L, L)] = acc

  pltpu.sync_copy(res_v, degp_hbm.at[cid, pl.ds(sid * CW, CW)])


def _sc_degree(dst_flat):
  return pl.kernel(
      _deg_body,
      out_type=jax.ShapeDtypeStruct((NC, NP), jnp.float32),
      mesh=_mesh(),
      compiler_params=pltpu.CompilerParams(needs_layout_passes=False),
      scratch_types=[
          pltpu.VMEM((EPW,), jnp.int32),
          pltpu.VMEM((NP,), jnp.float32),
          pltpu.VMEM((NS, NP // NS), jnp.float32),
          pltpu.VMEM((NP // NS,), jnp.float32),
          pltpu.VMEM_SHARED((NS, NP), jnp.float32),
      ],
  )(dst_flat)


def _edge_body(y_hbm, src_hbm, dst_hbm, z0_hbm, accp_hbm,
               src_v, dst_v, rows_v, acc_sh, sg, ss):
  cid = lax.axis_index("c")
  sid = lax.axis_index("s")
  wid = _worker_id()

  pltpu.sync_copy(z0_hbm.at[pl.ds(sid * RPT, RPT)],
                  acc_sh.at[pl.ds(sid * RPT, RPT)])
  plsc.subcore_barrier()

  def stage(ob):
    slot = lax.rem(ob, 2)
    pltpu.sync_copy(src_hbm.at[wid, pl.ds(ob * SB, SB)], src_v.at[slot])
    pltpu.sync_copy(dst_hbm.at[wid, pl.ds(ob * SB, SB)], dst_v.at[slot])

  def g_desc(j, b):
    return pltpu.make_async_copy(
        y_hbm.at[src_v.at[lax.rem(j // SB, 2), lax.rem(j, SB)]],
        rows_v.at[b], sg.at[b])

  def s_start(j, b):
    pltpu.async_copy(
        rows_v.at[b],
        acc_sh.at[dst_v.at[lax.rem(j // SB, 2), lax.rem(j, SB)]],
        ss.at[b], add=True)

  def s_wait(b):
    pltpu.make_async_copy(rows_v.at[b], acc_sh.at[pl.ds(0, K)],
                          ss.at[b]).wait()

  AH = RING // 2

  stage(0)
  for b in range(AH):
    g_desc(b, b).start()

  @pl.loop(0, NCHUNK // RING)
  def _group(q):
    for u in range(RING):
      j = q * RING + u
      g_desc(j, u).wait()
      s_start(j, u)
      jn = j + AH
      bn = (u + AH) % RING

      @pl.when(jn < NCHUNK)
      def _next():
        @pl.when(lax.rem(jn, SB) == 0)
        def _stage():
          stage(jn // SB)

        @pl.when(j >= AH)
        def _free():
          s_wait(bn)

        g_desc(jn, bn).start()

  for u in range(RING):
    s_wait((NCHUNK - RING + u) % RING)

  plsc.subcore_barrier()
  pltpu.sync_copy(acc_sh.at[pl.ds(sid * RPT, RPT)],
                  accp_hbm.at[cid, pl.ds(sid * RPT, RPT)])


def _sc_edge(y, src3, dst3, z0):
  return pl.kernel(
      _edge_body,
      out_type=jax.ShapeDtypeStruct((NC, NP, C), jnp.float32),
      mesh=_mesh(),
      compiler_params=pltpu.CompilerParams(needs_layout_passes=False),
      scratch_types=[
          pltpu.VMEM((2, SB, K), jnp.int32),
          pltpu.VMEM((2, SB, K), jnp.int32),
          pltpu.VMEM((RING, K, C), jnp.float32),
          pltpu.VMEM_SHARED((NP, C), jnp.float32),
          pltpu.SemaphoreType.DMA((RING,)),
          pltpu.SemaphoreType.DMA((RING,)),
      ],
  )(y, src3, dst3, z0)


def _decode_body(z_hbm, la_hbm, lb_hbm, scores_hbm,
                 ia_v, ib_v, ra_v, rb_v, s_v, sg):
  wid = _worker_id()
  NI = (NCH2 + NW - 1) // NW

  def stage_and_gather(i, slot):
    c = wid + i * NW

    @pl.when(c < NCH2)
    def _do():
      pltpu.sync_copy(la_hbm.at[c], ia_v.at[slot])
      pltpu.sync_copy(lb_hbm.at[c], ib_v.at[slot])
      pltpu.async_copy(z_hbm.at[ia_v.at[slot]], ra_v.at[slot], sg.at[slot])
      pltpu.async_copy(z_hbm.at[ib_v.at[slot]], rb_v.at[slot], sg.at[slot])

  def wait_rows(slot):
    pltpu.make_async_copy(z_hbm.at[pl.ds(0, K2)], ra_v.at[slot],
                          sg.at[slot]).wait()
    pltpu.make_async_copy(z_hbm.at[pl.ds(0, K2)], rb_v.at[slot],
                          sg.at[slot]).wait()

  stage_and_gather(0, 0)

  @pl.loop(0, NI)
  def _chunk(i):
    c = wid + i * NW
    slot = lax.rem(i, 2)
    stage_and_gather(i + 1, 1 - slot)

    @pl.when(c < NCH2)
    def _do():
      wait_rows(slot)

      @pl.loop(0, K2 // L)
      def _blk(jb):
        rows = jnp.arange(L, dtype=jnp.int32) + jb * L

        def _chan(l, acc):
          cols = jnp.full((L,), l, jnp.int32)
          va = plsc.load_gather(ra_v, [jnp.full((L,), slot, jnp.int32),
                                       rows, cols])
          vb = plsc.load_gather(rb_v, [jnp.full((L,), slot, jnp.int32),
                                       rows, cols])
          return acc + va * vb

        s_v[pl.ds(jb * L, L)] = lax.fori_loop(
            0, C, _chan, jnp.zeros((L,), jnp.float32), unroll=8)

      pltpu.sync_copy(s_v, scores_hbm.at[pl.ds(c * K2, K2)])


def _sc_decode(z, la2, lb2):
  return pl.kernel(
      _decode_body,
      out_type=jax.ShapeDtypeStruct((EL,), jnp.float32),
      mesh=_mesh(),
      compiler_params=pltpu.CompilerParams(needs_layout_passes=False),
      scratch_types=[
          pltpu.VMEM((2, K2), jnp.int32),
          pltpu.VMEM((2, K2), jnp.int32),
          pltpu.VMEM((2, K2, C), jnp.float32),
          pltpu.VMEM((2, K2, C), jnp.float32),
          pltpu.VMEM((K2,), jnp.float32),
          pltpu.SemaphoreType.DMA((2,)),
      ],
  )(z, la2, lb2)


B = 1000


def _dinv(degp_blk):
  deg = degp_blk[:, 0] + degp_blk[:, 1] + 1.0
  return lax.rsqrt(deg)


def _tc1_body(x_ref, w_ref, degp_ref, y_ref):
  dinv = _dinv(degp_ref[...])
  xw = jnp.dot(x_ref[...], w_ref[...], preferred_element_type=jnp.float32,
               precision=lax.Precision.HIGHEST)
  y_ref[...] = xw * dinv[:, None]


def _tc1(x, W1, degp):
  return pl.pallas_call(
      _tc1_body,
      grid=(N // B,),
      in_specs=[
          pl.BlockSpec((B, C), lambda i: (i, 0)),
          pl.BlockSpec((C, C), lambda i: (0, 0)),
          pl.BlockSpec((B, 2), lambda i: (i, 0)),
      ],
      out_specs=pl.BlockSpec((B, C), lambda i: (i, 0)),
      out_shape=jax.ShapeDtypeStruct((N, C), jnp.float32),
  )(x, W1, degp)


def _tc2_body(accp_ref, y1_ref, degp_ref, b1_ref, w_ref, y2_ref):
  dinv = _dinv(degp_ref[...])
  pre = (accp_ref[0] + accp_ref[1] + y1_ref[...]) * dinv[:, None] + b1_ref[0, :]
  h = jnp.maximum(pre, 0.0)
  xw = jnp.dot(h, w_ref[...], preferred_element_type=jnp.float32,
               precision=lax.Precision.HIGHEST)
  y2_ref[...] = xw * dinv[:, None]


def _tc2(accp, y1, degp, b1, W2):
  return pl.pallas_call(
      _tc2_body,
      grid=(N // B,),
      in_specs=[
          pl.BlockSpec((2, B, C), lambda i: (0, i, 0)),
          pl.BlockSpec((B, C), lambda i: (i, 0)),
          pl.BlockSpec((B, 2), lambda i: (i, 0)),
          pl.BlockSpec((1, C), lambda i: (0, 0)),
          pl.BlockSpec((C, C), lambda i: (0, 0)),
      ],
      out_specs=pl.BlockSpec((B, C), lambda i: (i, 0)),
      out_shape=jax.ShapeDtypeStruct((N, C), jnp.float32),
  )(accp, y1, degp, b1, W2)


def _tc3_body(accp_ref, y2_ref, degp_ref, b2_ref, z_ref):
  dinv = _dinv(degp_ref[...])
  z_ref[...] = ((accp_ref[0] + accp_ref[1] + y2_ref[...]) * dinv[:, None]
                + b2_ref[0, :])


def _tc3(accp, y2, degp, b2):
  return pl.pallas_call(
      _tc3_body,
      grid=(N // B,),
      in_specs=[
          pl.BlockSpec((2, B, C), lambda i: (0, i, 0)),
          pl.BlockSpec((B, C), lambda i: (i, 0)),
          pl.BlockSpec((B, 2), lambda i: (i, 0)),
          pl.BlockSpec((1, C), lambda i: (0, 0)),
      ],
      out_specs=pl.BlockSpec((B, C), lambda i: (i, 0)),
      out_shape=jax.ShapeDtypeStruct((N, C), jnp.float32),
  )(accp, y2, degp, b2)


def kernel(x, edge_index, edge_label_index, W1, b1, W2, b2):
  ei = edge_index.astype(jnp.int32)
  el = edge_label_index.astype(jnp.int32)
  src3 = ei[0].reshape(NW, NCHUNK, K)
  dst3 = ei[1].reshape(NW, NCHUNK, K)
  dst_flat = ei[1]
  la2 = el[0].reshape(NCH2, K2)
  lb2 = el[1].reshape(NCH2, K2)
  b1r = b1.reshape(1, C)
  b2r = b2.reshape(1, C)

  degp = _sc_degree(dst_flat)[:, :N].T
  y1 = _tc1(x, W1, degp)
  z0 = jnp.zeros((NP, C), jnp.float32)
  accp1 = _sc_edge(y1, src3, dst3, z0)[:, :N]
  y2 = _tc2(accp1, y1, degp, b1r, W2)
  accp2 = _sc_edge(y2, src3, dst3, z0)[:, :N]
  z = _tc3(accp2, y2, degp, b2r)
  scores = _sc_decode(z, la2, lb2)
  return scores


# speedup vs baseline: 48.7282x; 1.1139x over previous
import functools

import jax
import jax.numpy as jnp
from jax import lax
from jax.experimental import pallas as pl
from jax.experimental.pallas import tpu as pltpu
from jax.experimental.pallas import tpu_sc as plsc

N = 10000
E = 320000
EL = 20000
C = 128
NC, NS, L = 2, 16, 16
NW = NC * NS
EPW = E // NW
K = 25
NCHUNK = EPW // K
SB = 8
RING = 8
NP = 10240
HR = NP // L
RPT = NP // NS
K2 = 80
NCH2 = EL // K2

_mesh = functools.partial(
    plsc.VectorSubcoreMesh, core_axis_name="c", subcore_axis_name="s",
    num_cores=NC, num_subcores=NS)


def _worker_id():
  return lax.axis_index("s") * NC + lax.axis_index("c")


def _deg_body(dst_hbm, degp_hbm, idx_v, hist_v, buf_v, res_v, shared_v):
  cid = lax.axis_index("c")
  sid = lax.axis_index("s")
  wid = _worker_id()
  CW = NP // NS

  @pl.loop(0, NP // L)
  def _zero(i):
    hist_v[pl.ds(i * L, L)] = jnp.zeros((L,), jnp.float32)

  pltpu.sync_copy(dst_hbm.at[pl.ds(wid * EPW, EPW)], idx_v)

  @pl.loop(0, EPW // L)
  def _acc(i):
    idx = idx_v[pl.ds(i * L, L)]
    plsc.addupdate_scatter(hist_v, [idx], jnp.ones((L,), jnp.float32))

  pltpu.sync_copy(hist_v, shared_v.at[sid])
  plsc.subcore_barrier()
  for r in range(NS):
    pltpu.sync_copy(shared_v.at[r, pl.ds(sid * CW, CW)], buf_v.at[r])

  @pl.loop(0, CW // L)
  def _sum(i):
    acc = jnp.zeros((L,), jnp.float32)
    for r in range(NS):
      acc += buf_v[r, pl.ds(i * L, L)]
    res_v[pl.ds(i * L, L)] = acc

  pltpu.sync_copy(res_v, degp_hbm.at[cid, pl.ds(sid * CW, CW)])


def _sc_degree(dst_flat):
  return pl.kernel(
      _deg_body,
      out_type=jax.ShapeDtypeStruct((NC, NP), jnp.float32),
      mesh=_mesh(),
      compiler_params=pltpu.CompilerParams(needs_layout_passes=False),
      scratch_types=[
          pltpu.VMEM((EPW,), jnp.int32),
          pltpu.VMEM((NP,), jnp.float32),
          pltpu.VMEM((NS, NP // NS), jnp.float32),
          pltpu.VMEM((NP // NS,), jnp.float32),
          pltpu.VMEM_SHARED((NS, NP), jnp.float32),
      ],
  )(dst_flat)


def _edge_body(y_hbm, src_hbm, dst_hbm, z0_hbm, accp_hbm,
               src_v, dst_v, rows_v, acc_sh, sg, ss):
  cid = lax.axis_index("c")
  sid = lax.axis_index("s")
  wid = _worker_id()

  pltpu.sync_copy(z0_hbm.at[pl.ds(sid * RPT, RPT)],
                  acc_sh.at[pl.ds(sid * RPT, RPT)])
  plsc.subcore_barrier()

  def stage(ob):
    slot = lax.rem(ob, 2)
    pltpu.sync_copy(src_hbm.at[wid, pl.ds(ob * SB, SB)], src_v.at[slot])
    pltpu.sync_copy(dst_hbm.at[wid, pl.ds(ob * SB, SB)], dst_v.at[slot])

  def g_desc(j, b):
    return pltpu.make_async_copy(
        y_hbm.at[src_v.at[lax.rem(j // SB, 2), lax.rem(j, SB)]],
        rows_v.at[b], sg.at[b])

  def s_start(j, b):
    pltpu.async_copy(
        rows_v.at[b],
        acc_sh.at[dst_v.at[lax.rem(j // SB, 2), lax.rem(j, SB)]],
        ss.at[b], add=True)

  def s_wait(b):
    pltpu.make_async_copy(rows_v.at[b], acc_sh.at[pl.ds(0, K)],
                          ss.at[b]).wait()

  AH = RING // 2

  stage(0)
  for b in range(AH):
    g_desc(b, b).start()

  @pl.loop(0, NCHUNK // RING)
  def _group(q):
    for u in range(RING):
      j = q * RING + u
      g_desc(j, u).wait()
      s_start(j, u)
      jn = j + AH
      bn = (u + AH) % RING

      @pl.when(jn < NCHUNK)
      def _next():
        @pl.when(lax.rem(jn, SB) == 0)
        def _stage():
          stage(jn // SB)

        @pl.when(j >= AH)
        def _free():
          s_wait(bn)

        g_desc(jn, bn).start()

  for u in range(RING):
    s_wait((NCHUNK - RING + u) % RING)

  plsc.subcore_barrier()
  pltpu.sync_copy(acc_sh.at[pl.ds(sid * RPT, RPT)],
                  accp_hbm.at[cid, pl.ds(sid * RPT, RPT)])


def _sc_edge(y, src3, dst3, z0):
  return pl.kernel(
      _edge_body,
      out_type=jax.ShapeDtypeStruct((NC, NP, C), jnp.float32),
      mesh=_mesh(),
      compiler_params=pltpu.CompilerParams(needs_layout_passes=False),
      scratch_types=[
          pltpu.VMEM((2, SB, K), jnp.int32),
          pltpu.VMEM((2, SB, K), jnp.int32),
          pltpu.VMEM((RING, K, C), jnp.float32),
          pltpu.VMEM_SHARED((NP, C), jnp.float32),
          pltpu.SemaphoreType.DMA((RING,)),
          pltpu.SemaphoreType.DMA((RING,)),
      ],
  )(y, src3, dst3, z0)


def _decode_body(z_hbm, la_hbm, lb_hbm, za_hbm, zb_hbm,
                 ia_v, ib_v, ra_v, rb_v, sg):
  wid = _worker_id()
  NI = (NCH2 + NW - 1) // NW

  def stage_and_gather(i, slot):
    c = wid + i * NW

    @pl.when(c < NCH2)
    def _do():
      pltpu.sync_copy(la_hbm.at[c], ia_v.at[slot])
      pltpu.sync_copy(lb_hbm.at[c], ib_v.at[slot])
      pltpu.async_copy(z_hbm.at[ia_v.at[slot]], ra_v.at[slot], sg.at[slot])
      pltpu.async_copy(z_hbm.at[ib_v.at[slot]], rb_v.at[slot], sg.at[slot])

  def wait_rows(slot):
    pltpu.make_async_copy(z_hbm.at[pl.ds(0, K2)], ra_v.at[slot],
                          sg.at[slot]).wait()
    pltpu.make_async_copy(z_hbm.at[pl.ds(0, K2)], rb_v.at[slot],
                          sg.at[slot]).wait()

  stage_and_gather(0, 0)

  @pl.loop(0, NI)
  def _chunk(i):
    c = wid + i * NW
    slot = lax.rem(i, 2)
    stage_and_gather(i + 1, 1 - slot)

    @pl.when(c < NCH2)
    def _do():
      wait_rows(slot)
      pltpu.sync_copy(ra_v.at[slot], za_hbm.at[pl.ds(c * K2, K2)])
      pltpu.sync_copy(rb_v.at[slot], zb_hbm.at[pl.ds(c * K2, K2)])


def _sc_decode_gather(z, la2, lb2):
  return pl.kernel(
      _decode_body,
      out_type=(jax.ShapeDtypeStruct((EL, C), jnp.float32),
                jax.ShapeDtypeStruct((EL, C), jnp.float32)),
      mesh=_mesh(),
      compiler_params=pltpu.CompilerParams(needs_layout_passes=False),
      scratch_types=[
          pltpu.VMEM((2, K2), jnp.int32),
          pltpu.VMEM((2, K2), jnp.int32),
          pltpu.VMEM((2, K2, C), jnp.float32),
          pltpu.VMEM((2, K2, C), jnp.float32),
          pltpu.SemaphoreType.DMA((2,)),
      ],
  )(z, la2, lb2)


B = 1000


def _dinv(degp_blk):
  deg = degp_blk[:, 0] + degp_blk[:, 1] + 1.0
  return lax.rsqrt(deg)


def _tc1_body(x_ref, w_ref, degp_ref, y_ref):
  dinv = _dinv(degp_ref[...])
  xw = jnp.dot(x_ref[...], w_ref[...], preferred_element_type=jnp.float32,
               precision=lax.Precision.HIGHEST)
  y_ref[...] = xw * dinv[:, None]


def _tc1(x, W1, degp):
  return pl.pallas_call(
      _tc1_body,
      grid=(N // B,),
      in_specs=[
          pl.BlockSpec((B, C), lambda i: (i, 0)),
          pl.BlockSpec((C, C), lambda i: (0, 0)),
          pl.BlockSpec((B, 2), lambda i: (i, 0)),
      ],
      out_specs=pl.BlockSpec((B, C), lambda i: (i, 0)),
      out_shape=jax.ShapeDtypeStruct((N, C), jnp.float32),
  )(x, W1, degp)


def _tc2_body(accp_ref, y1_ref, degp_ref, b1_ref, w_ref, y2_ref):
  dinv = _dinv(degp_ref[...])
  pre = (accp_ref[0] + accp_ref[1] + y1_ref[...]) * dinv[:, None] + b1_ref[0, :]
  h = jnp.maximum(pre, 0.0)
  xw = jnp.dot(h, w_ref[...], preferred_element_type=jnp.float32,
               precision=lax.Precision.HIGHEST)
  y2_ref[...] = xw * dinv[:, None]


def _tc2(accp, y1, degp, b1, W2):
  return pl.pallas_call(
      _tc2_body,
      grid=(N // B,),
      in_specs=[
          pl.BlockSpec((2, B, C), lambda i: (0, i, 0)),
          pl.BlockSpec((B, C), lambda i: (i, 0)),
          pl.BlockSpec((B, 2), lambda i: (i, 0)),
          pl.BlockSpec((1, C), lambda i: (0, 0)),
          pl.BlockSpec((C, C), lambda i: (0, 0)),
      ],
      out_specs=pl.BlockSpec((B, C), lambda i: (i, 0)),
      out_shape=jax.ShapeDtypeStruct((N, C), jnp.float32),
  )(accp, y1, degp, b1, W2)


def _tc3_body(accp_ref, y2_ref, degp_ref, b2_ref, z_ref):
  dinv = _dinv(degp_ref[...])
  z_ref[...] = ((accp_ref[0] + accp_ref[1] + y2_ref[...]) * dinv[:, None]
                + b2_ref[0, :])


def _tc3(accp, y2, degp, b2):
  return pl.pallas_call(
      _tc3_body,
      grid=(N // B,),
      in_specs=[
          pl.BlockSpec((2, B, C), lambda i: (0, i, 0)),
          pl.BlockSpec((B, C), lambda i: (i, 0)),
          pl.BlockSpec((B, 2), lambda i: (i, 0)),
          pl.BlockSpec((1, C), lambda i: (0, 0)),
      ],
      out_specs=pl.BlockSpec((B, C), lambda i: (i, 0)),
      out_shape=jax.ShapeDtypeStruct((N, C), jnp.float32),
  )(accp, y2, degp, b2)


def _tc4_body(za_ref, zb_ref, s_ref):
  s_ref[...] = jnp.sum(za_ref[...] * zb_ref[...], axis=1, keepdims=True)


def _tc4(za, zb):
  BD = 2000
  return pl.pallas_call(
      _tc4_body,
      grid=(EL // BD,),
      in_specs=[
          pl.BlockSpec((BD, C), lambda i: (i, 0)),
          pl.BlockSpec((BD, C), lambda i: (i, 0)),
      ],
      out_specs=pl.BlockSpec((BD, 1), lambda i: (i, 0)),
      out_shape=jax.ShapeDtypeStruct((EL, 1), jnp.float32),
  )(za, zb)


def kernel(x, edge_index, edge_label_index, W1, b1, W2, b2):
  ei = edge_index.astype(jnp.int32)
  el = edge_label_index.astype(jnp.int32)
  src3 = ei[0].reshape(NW, NCHUNK, K)
  dst3 = ei[1].reshape(NW, NCHUNK, K)
  dst_flat = ei[1]
  la2 = el[0].reshape(NCH2, K2)
  lb2 = el[1].reshape(NCH2, K2)
  b1r = b1.reshape(1, C)
  b2r = b2.reshape(1, C)

  degp = _sc_degree(dst_flat).T
  y1 = _tc1(x, W1, degp)
  z0 = jnp.zeros((NP, C), jnp.float32)
  accp1 = _sc_edge(y1, src3, dst3, z0)
  y2 = _tc2(accp1, y1, degp, b1r, W2)
  accp2 = _sc_edge(y2, src3, dst3, z0)
  z = _tc3(accp2, y2, degp, b2r)
  za, zb = _sc_decode_gather(z, la2, lb2)
  scores = _tc4(za, zb).reshape(EL)
  return scores


# speedup vs baseline: 54.4580x; 1.1176x over previous
import functools

import jax
import jax.numpy as jnp
from jax import lax
from jax.experimental import pallas as pl
from jax.experimental.pallas import tpu as pltpu
from jax.experimental.pallas import tpu_sc as plsc

N = 10000
E = 320000
EL = 20000
C = 128
NC, NS, L = 2, 16, 16
NW = NC * NS
EPW = E // NW
K = 50
NCHUNK = EPW // K
SB = 8
RING = 4
NP = 10240
HR = NP // L
RPT = NP // NS
K2 = 80
NCH2 = EL // K2

_mesh = functools.partial(
    plsc.VectorSubcoreMesh, core_axis_name="c", subcore_axis_name="s",
    num_cores=NC, num_subcores=NS)


def _worker_id():
  return lax.axis_index("s") * NC + lax.axis_index("c")


def _deg_body(dst_hbm, degp_hbm, idx_v, hist_v, buf_v, res_v, shared_v):
  cid = lax.axis_index("c")
  sid = lax.axis_index("s")
  wid = _worker_id()
  CW = NP // NS

  @pl.loop(0, NP // L)
  def _zero(i):
    hist_v[pl.ds(i * L, L)] = jnp.zeros((L,), jnp.float32)

  pltpu.sync_copy(dst_hbm.at[pl.ds(wid * EPW, EPW)], idx_v)

  @pl.loop(0, EPW // L)
  def _acc(i):
    idx = idx_v[pl.ds(i * L, L)]
    plsc.addupdate_scatter(hist_v, [idx], jnp.ones((L,), jnp.float32))

  pltpu.sync_copy(hist_v, shared_v.at[sid])
  plsc.subcore_barrier()
  for r in range(NS):
    pltpu.sync_copy(shared_v.at[r, pl.ds(sid * CW, CW)], buf_v.at[r])

  @pl.loop(0, CW // L)
  def _sum(i):
    acc = jnp.zeros((L,), jnp.float32)
    for r in range(NS):
      acc += buf_v[r, pl.ds(i * L, L)]
    res_v[pl.ds(i * L, L)] = acc

  pltpu.sync_copy(res_v, degp_hbm.at[cid, pl.ds(sid * CW, CW)])


def _sc_degree(dst_flat):
  return pl.kernel(
      _deg_body,
      out_type=jax.ShapeDtypeStruct((NC, NP), jnp.float32),
      mesh=_mesh(),
      compiler_params=pltpu.CompilerParams(needs_layout_passes=False),
      scratch_types=[
          pltpu.VMEM((EPW,), jnp.int32),
          pltpu.VMEM((NP,), jnp.float32),
          pltpu.VMEM((NS, NP // NS), jnp.float32),
          pltpu.VMEM((NP // NS,), jnp.float32),
          pltpu.VMEM_SHARED((NS, NP), jnp.float32),
      ],
  )(dst_flat)


def _edge_body(y_hbm, src_hbm, dst_hbm, z0_hbm, accp_hbm,
               src_v, dst_v, rows_v, acc_sh, sg, ss):
  cid = lax.axis_index("c")
  sid = lax.axis_index("s")
  wid = _worker_id()

  pltpu.sync_copy(z0_hbm.at[pl.ds(sid * RPT, RPT)],
                  acc_sh.at[pl.ds(sid * RPT, RPT)])
  plsc.subcore_barrier()

  def stage(ob):
    slot = lax.rem(ob, 2)
    pltpu.sync_copy(src_hbm.at[wid, pl.ds(ob * SB, SB)], src_v.at[slot])
    pltpu.sync_copy(dst_hbm.at[wid, pl.ds(ob * SB, SB)], dst_v.at[slot])

  def g_desc(j, b):
    return pltpu.make_async_copy(
        y_hbm.at[src_v.at[lax.rem(j // SB, 2), lax.rem(j, SB)]],
        rows_v.at[b], sg.at[b])

  def s_start(j, b):
    pltpu.async_copy(
        rows_v.at[b],
        acc_sh.at[dst_v.at[lax.rem(j // SB, 2), lax.rem(j, SB)]],
        ss.at[b], add=True)

  def s_wait(b):
    pltpu.make_async_copy(rows_v.at[b], acc_sh.at[pl.ds(0, K)],
                          ss.at[b]).wait()

  AH = RING // 2

  stage(0)
  for b in range(AH):
    g_desc(b, b).start()

  @pl.loop(0, NCHUNK // RING)
  def _group(q):
    for u in range(RING):
      j = q * RING + u
      g_desc(j, u).wait()
      s_start(j, u)
      jn = j + AH
      bn = (u + AH) % RING

      @pl.when(jn < NCHUNK)
      def _next():
        @pl.when(lax.rem(jn, SB) == 0)
        def _stage():
          stage(jn // SB)

        @pl.when(j >= AH)
        def _free():
          s_wait(bn)

        g_desc(jn, bn).start()

  for u in range(RING):
    s_wait((NCHUNK - RING + u) % RING)

  plsc.subcore_barrier()
  pltpu.sync_copy(acc_sh.at[pl.ds(sid * RPT, RPT)],
                  accp_hbm.at[cid, pl.ds(sid * RPT, RPT)])


def _sc_edge(y, src3, dst3, z0):
  return pl.kernel(
      _edge_body,
      out_type=jax.ShapeDtypeStruct((NC, NP, C), jnp.float32),
      mesh=_mesh(),
      compiler_params=pltpu.CompilerParams(needs_layout_passes=False),
      scratch_types=[
          pltpu.VMEM((2, SB, K), jnp.int32),
          pltpu.VMEM((2, SB, K), jnp.int32),
          pltpu.VMEM((RING, K, C), jnp.float32),
          pltpu.VMEM_SHARED((NP, C), jnp.float32),
          pltpu.SemaphoreType.DMA((RING,)),
          pltpu.SemaphoreType.DMA((RING,)),
      ],
  )(y, src3, dst3, z0)


def _decode_body(z_hbm, la_hbm, lb_hbm, za_hbm, zb_hbm,
                 ia_v, ib_v, ra_v, rb_v, sg):
  wid = _worker_id()
  NI = (NCH2 + NW - 1) // NW

  def stage_and_gather(i, slot):
    c = wid + i * NW

    @pl.when(c < NCH2)
    def _do():
      pltpu.sync_copy(la_hbm.at[c], ia_v.at[slot])
      pltpu.sync_copy(lb_hbm.at[c], ib_v.at[slot])
      pltpu.async_copy(z_hbm.at[ia_v.at[slot]], ra_v.at[slot], sg.at[slot])
      pltpu.async_copy(z_hbm.at[ib_v.at[slot]], rb_v.at[slot], sg.at[slot])

  def wait_rows(slot):
    pltpu.make_async_copy(z_hbm.at[pl.ds(0, K2)], ra_v.at[slot],
                          sg.at[slot]).wait()
    pltpu.make_async_copy(z_hbm.at[pl.ds(0, K2)], rb_v.at[slot],
                          sg.at[slot]).wait()

  stage_and_gather(0, 0)

  @pl.loop(0, NI)
  def _chunk(i):
    c = wid + i * NW
    slot = lax.rem(i, 2)
    stage_and_gather(i + 1, 1 - slot)

    @pl.when(c < NCH2)
    def _do():
      wait_rows(slot)
      pltpu.sync_copy(ra_v.at[slot], za_hbm.at[pl.ds(c * K2, K2)])
      pltpu.sync_copy(rb_v.at[slot], zb_hbm.at[pl.ds(c * K2, K2)])


def _sc_decode_gather(z, la2, lb2):
  return pl.kernel(
      _decode_body,
      out_type=(jax.ShapeDtypeStruct((EL, C), jnp.float32),
                jax.ShapeDtypeStruct((EL, C), jnp.float32)),
      mesh=_mesh(),
      compiler_params=pltpu.CompilerParams(needs_layout_passes=False),
      scratch_types=[
          pltpu.VMEM((2, K2), jnp.int32),
          pltpu.VMEM((2, K2), jnp.int32),
          pltpu.VMEM((2, K2, C), jnp.float32),
          pltpu.VMEM((2, K2, C), jnp.float32),
          pltpu.SemaphoreType.DMA((2,)),
      ],
  )(z, la2, lb2)


B = 1000


def _dinv(degp_blk):
  deg = degp_blk[:, 0] + degp_blk[:, 1] + 1.0
  return lax.rsqrt(deg)


def _tc1_body(x_ref, w_ref, degp_ref, y_ref):
  dinv = _dinv(degp_ref[...])
  xw = jnp.dot(x_ref[...], w_ref[...], preferred_element_type=jnp.float32,
               precision=lax.Precision.HIGHEST)
  y_ref[...] = xw * dinv[:, None]


def _tc1(x, W1, degp):
  return pl.pallas_call(
      _tc1_body,
      grid=(N // B,),
      in_specs=[
          pl.BlockSpec((B, C), lambda i: (i, 0)),
          pl.BlockSpec((C, C), lambda i: (0, 0)),
          pl.BlockSpec((B, 2), lambda i: (i, 0)),
      ],
      out_specs=pl.BlockSpec((B, C), lambda i: (i, 0)),
      out_shape=jax.ShapeDtypeStruct((N, C), jnp.float32),
  )(x, W1, degp)


def _tc2_body(accp_ref, y1_ref, degp_ref, b1_ref, w_ref, y2_ref):
  dinv = _dinv(degp_ref[...])
  pre = (accp_ref[0] + accp_ref[1] + y1_ref[...]) * dinv[:, None] + b1_ref[0, :]
  h = jnp.maximum(pre, 0.0)
  xw = jnp.dot(h, w_ref[...], preferred_element_type=jnp.float32,
               precision=lax.Precision.HIGHEST)
  y2_ref[...] = xw * dinv[:, None]


def _tc2(accp, y1, degp, b1, W2):
  return pl.pallas_call(
      _tc2_body,
      grid=(N // B,),
      in_specs=[
          pl.BlockSpec((2, B, C), lambda i: (0, i, 0)),
          pl.BlockSpec((B, C), lambda i: (i, 0)),
          pl.BlockSpec((B, 2), lambda i: (i, 0)),
          pl.BlockSpec((1, C), lambda i: (0, 0)),
          pl.BlockSpec((C, C), lambda i: (0, 0)),
      ],
      out_specs=pl.BlockSpec((B, C), lambda i: (i, 0)),
      out_shape=jax.ShapeDtypeStruct((N, C), jnp.float32),
  )(accp, y1, degp, b1, W2)


def _tc3_body(accp_ref, y2_ref, degp_ref, b2_ref, z_ref):
  dinv = _dinv(degp_ref[...])
  z_ref[...] = ((accp_ref[0] + accp_ref[1] + y2_ref[...]) * dinv[:, None]
                + b2_ref[0, :])


def _tc3(accp, y2, degp, b2):
  return pl.pallas_call(
      _tc3_body,
      grid=(N // B,),
      in_specs=[
          pl.BlockSpec((2, B, C), lambda i: (0, i, 0)),
          pl.BlockSpec((B, C), lambda i: (i, 0)),
          pl.BlockSpec((B, 2), lambda i: (i, 0)),
          pl.BlockSpec((1, C), lambda i: (0, 0)),
      ],
      out_specs=pl.BlockSpec((B, C), lambda i: (i, 0)),
      out_shape=jax.ShapeDtypeStruct((N, C), jnp.float32),
  )(accp, y2, degp, b2)


def _tc4_body(za_ref, zb_ref, s_ref):
  s_ref[...] = jnp.sum(za_ref[...] * zb_ref[...], axis=1, keepdims=True)


def _tc4(za, zb):
  BD = 2000
  return pl.pallas_call(
      _tc4_body,
      grid=(EL // BD,),
      in_specs=[
          pl.BlockSpec((BD, C), lambda i: (i, 0)),
          pl.BlockSpec((BD, C), lambda i: (i, 0)),
      ],
      out_specs=pl.BlockSpec((BD, 1), lambda i: (i, 0)),
      out_shape=jax.ShapeDtypeStruct((EL, 1), jnp.float32),
  )(za, zb)


def kernel(x, edge_index, edge_label_index, W1, b1, W2, b2):
  ei = edge_index.astype(jnp.int32)
  el = edge_label_index.astype(jnp.int32)
  src3 = ei[0].reshape(NW, NCHUNK, K)
  dst3 = ei[1].reshape(NW, NCHUNK, K)
  dst_flat = ei[1]
  la2 = el[0].reshape(NCH2, K2)
  lb2 = el[1].reshape(NCH2, K2)
  b1r = b1.reshape(1, C)
  b2r = b2.reshape(1, C)

  degp = _sc_degree(dst_flat).T
  y1 = _tc1(x, W1, degp)
  z0 = jnp.zeros((NP, C), jnp.float32)
  accp1 = _sc_edge(y1, src3, dst3, z0)
  y2 = _tc2(accp1, y1, degp, b1r, W2)
  accp2 = _sc_edge(y2, src3, dst3, z0)
  z = _tc3(accp2, y2, degp, b2r)
  za, zb = _sc_decode_gather(z, la2, lb2)
  scores = _tc4(za, zb).reshape(EL)
  return scores


# speedup vs baseline: 58.4878x; 1.0740x over previous
import functools

import jax
import jax.numpy as jnp
from jax import lax
from jax.experimental import pallas as pl
from jax.experimental.pallas import tpu as pltpu
from jax.experimental.pallas import tpu_sc as plsc

N = 10000
E = 320000
EL = 20000
C = 128
NC, NS, L = 2, 16, 16
NW = NC * NS
EPW = E // NW
K = 125
NCHUNK = EPW // K
SB = 8
RING = 2
NP = 10240
HR = NP // L
RPT = NP // NS
K2 = 80
NCH2 = EL // K2

_mesh = functools.partial(
    plsc.VectorSubcoreMesh, core_axis_name="c", subcore_axis_name="s",
    num_cores=NC, num_subcores=NS)


def _worker_id():
  return lax.axis_index("s") * NC + lax.axis_index("c")


def _deg_body(dst_hbm, degp_hbm, idx_v, hist_v, buf_v, res_v, shared_v):
  cid = lax.axis_index("c")
  sid = lax.axis_index("s")
  wid = _worker_id()
  CW = NP // NS

  @pl.loop(0, NP // L)
  def _zero(i):
    hist_v[pl.ds(i * L, L)] = jnp.zeros((L,), jnp.float32)

  pltpu.sync_copy(dst_hbm.at[pl.ds(wid * EPW, EPW)], idx_v)

  @pl.loop(0, EPW // L)
  def _acc(i):
    idx = idx_v[pl.ds(i * L, L)]
    plsc.addupdate_scatter(hist_v, [idx], jnp.ones((L,), jnp.float32))

  pltpu.sync_copy(hist_v, shared_v.at[sid])
  plsc.subcore_barrier()
  for r in range(NS):
    pltpu.sync_copy(shared_v.at[r, pl.ds(sid * CW, CW)], buf_v.at[r])

  @pl.loop(0, CW // L)
  def _sum(i):
    acc = jnp.zeros((L,), jnp.float32)
    for r in range(NS):
      acc += buf_v[r, pl.ds(i * L, L)]
    res_v[pl.ds(i * L, L)] = acc

  pltpu.sync_copy(res_v, degp_hbm.at[cid, pl.ds(sid * CW, CW)])


def _sc_degree(dst_flat):
  return pl.kernel(
      _deg_body,
      out_type=jax.ShapeDtypeStruct((NC, NP), jnp.float32),
      mesh=_mesh(),
      compiler_params=pltpu.CompilerParams(needs_layout_passes=False),
      scratch_types=[
          pltpu.VMEM((EPW,), jnp.int32),
          pltpu.VMEM((NP,), jnp.float32),
          pltpu.VMEM((NS, NP // NS), jnp.float32),
          pltpu.VMEM((NP // NS,), jnp.float32),
          pltpu.VMEM_SHARED((NS, NP), jnp.float32),
      ],
  )(dst_flat)


def _edge_body(y_hbm, src_hbm, dst_hbm, z0_hbm, accp_hbm,
               src_v, dst_v, rows_v, acc_sh, sg, ss):
  cid = lax.axis_index("c")
  sid = lax.axis_index("s")
  wid = _worker_id()

  pltpu.sync_copy(z0_hbm.at[pl.ds(sid * RPT, RPT)],
                  acc_sh.at[pl.ds(sid * RPT, RPT)])
  plsc.subcore_barrier()

  def stage(ob):
    slot = lax.rem(ob, 2)
    pltpu.sync_copy(src_hbm.at[wid, pl.ds(ob * SB, SB)], src_v.at[slot])
    pltpu.sync_copy(dst_hbm.at[wid, pl.ds(ob * SB, SB)], dst_v.at[slot])

  def g_desc(j, b):
    return pltpu.make_async_copy(
        y_hbm.at[src_v.at[lax.rem(j // SB, 2), lax.rem(j, SB)]],
        rows_v.at[b], sg.at[b])

  def s_start(j, b):
    pltpu.async_copy(
        rows_v.at[b],
        acc_sh.at[dst_v.at[lax.rem(j // SB, 2), lax.rem(j, SB)]],
        ss.at[b], add=True)

  def s_wait(b):
    pltpu.make_async_copy(rows_v.at[b], acc_sh.at[pl.ds(0, K)],
                          ss.at[b]).wait()

  AH = RING // 2

  stage(0)
  for b in range(AH):
    g_desc(b, b).start()

  @pl.loop(0, NCHUNK // RING)
  def _group(q):
    for u in range(RING):
      j = q * RING + u
      g_desc(j, u).wait()
      s_start(j, u)
      jn = j + AH
      bn = (u + AH) % RING

      @pl.when(jn < NCHUNK)
      def _next():
        @pl.when(lax.rem(jn, SB) == 0)
        def _stage():
          stage(jn // SB)

        @pl.when(j >= AH)
        def _free():
          s_wait(bn)

        g_desc(jn, bn).start()

  for u in range(RING):
    s_wait((NCHUNK - RING + u) % RING)

  plsc.subcore_barrier()
  pltpu.sync_copy(acc_sh.at[pl.ds(sid * RPT, RPT)],
                  accp_hbm.at[cid, pl.ds(sid * RPT, RPT)])


def _sc_edge(y, src3, dst3, z0):
  return pl.kernel(
      _edge_body,
      out_type=jax.ShapeDtypeStruct((NC, NP, C), jnp.float32),
      mesh=_mesh(),
      compiler_params=pltpu.CompilerParams(needs_layout_passes=False),
      scratch_types=[
          pltpu.VMEM((2, SB, K), jnp.int32),
          pltpu.VMEM((2, SB, K), jnp.int32),
          pltpu.VMEM((RING, K, C), jnp.float32),
          pltpu.VMEM_SHARED((NP, C), jnp.float32),
          pltpu.SemaphoreType.DMA((RING,)),
          pltpu.SemaphoreType.DMA((RING,)),
      ],
  )(y, src3, dst3, z0)


def _decode_body(z_hbm, la_hbm, lb_hbm, za_hbm, zb_hbm,
                 ia_v, ib_v, ra_v, rb_v, sg):
  wid = _worker_id()
  NI = (NCH2 + NW - 1) // NW

  def stage_and_gather(i, slot):
    c = wid + i * NW

    @pl.when(c < NCH2)
    def _do():
      pltpu.sync_copy(la_hbm.at[c], ia_v.at[slot])
      pltpu.sync_copy(lb_hbm.at[c], ib_v.at[slot])
      pltpu.async_copy(z_hbm.at[ia_v.at[slot]], ra_v.at[slot], sg.at[slot])
      pltpu.async_copy(z_hbm.at[ib_v.at[slot]], rb_v.at[slot], sg.at[slot])

  def wait_rows(slot):
    pltpu.make_async_copy(z_hbm.at[pl.ds(0, K2)], ra_v.at[slot],
                          sg.at[slot]).wait()
    pltpu.make_async_copy(z_hbm.at[pl.ds(0, K2)], rb_v.at[slot],
                          sg.at[slot]).wait()

  stage_and_gather(0, 0)

  @pl.loop(0, NI)
  def _chunk(i):
    c = wid + i * NW
    slot = lax.rem(i, 2)
    stage_and_gather(i + 1, 1 - slot)

    @pl.when(c < NCH2)
    def _do():
      wait_rows(slot)
      pltpu.sync_copy(ra_v.at[slot], za_hbm.at[pl.ds(c * K2, K2)])
      pltpu.sync_copy(rb_v.at[slot], zb_hbm.at[pl.ds(c * K2, K2)])


def _sc_decode_gather(z, la2, lb2):
  return pl.kernel(
      _decode_body,
      out_type=(jax.ShapeDtypeStruct((EL, C), jnp.float32),
                jax.ShapeDtypeStruct((EL, C), jnp.float32)),
      mesh=_mesh(),
      compiler_params=pltpu.CompilerParams(needs_layout_passes=False),
      scratch_types=[
          pltpu.VMEM((2, K2), jnp.int32),
          pltpu.VMEM((2, K2), jnp.int32),
          pltpu.VMEM((2, K2, C), jnp.float32),
          pltpu.VMEM((2, K2, C), jnp.float32),
          pltpu.SemaphoreType.DMA((2,)),
      ],
  )(z, la2, lb2)


B = 1000


def _dinv(degp_blk):
  deg = degp_blk[:, 0] + degp_blk[:, 1] + 1.0
  return lax.rsqrt(deg)


def _tc1_body(x_ref, w_ref, degp_ref, y_ref):
  dinv = _dinv(degp_ref[...])
  xw = jnp.dot(x_ref[...], w_ref[...], preferred_element_type=jnp.float32,
               precision=lax.Precision.HIGHEST)
  y_ref[...] = xw * dinv[:, None]


def _tc1(x, W1, degp):
  return pl.pallas_call(
      _tc1_body,
      grid=(N // B,),
      in_specs=[
          pl.BlockSpec((B, C), lambda i: (i, 0)),
          pl.BlockSpec((C, C), lambda i: (0, 0)),
          pl.BlockSpec((B, 2), lambda i: (i, 0)),
      ],
      out_specs=pl.BlockSpec((B, C), lambda i: (i, 0)),
      out_shape=jax.ShapeDtypeStruct((N, C), jnp.float32),
  )(x, W1, degp)


def _tc2_body(accp_ref, y1_ref, degp_ref, b1_ref, w_ref, y2_ref):
  dinv = _dinv(degp_ref[...])
  pre = (accp_ref[0] + accp_ref[1] + y1_ref[...]) * dinv[:, None] + b1_ref[0, :]
  h = jnp.maximum(pre, 0.0)
  xw = jnp.dot(h, w_ref[...], preferred_element_type=jnp.float32,
               precision=lax.Precision.HIGHEST)
  y2_ref[...] = xw * dinv[:, None]


def _tc2(accp, y1, degp, b1, W2):
  return pl.pallas_call(
      _tc2_body,
      grid=(N // B,),
      in_specs=[
          pl.BlockSpec((2, B, C), lambda i: (0, i, 0)),
          pl.BlockSpec((B, C), lambda i: (i, 0)),
          pl.BlockSpec((B, 2), lambda i: (i, 0)),
          pl.BlockSpec((1, C), lambda i: (0, 0)),
          pl.BlockSpec((C, C), lambda i: (0, 0)),
      ],
      out_specs=pl.BlockSpec((B, C), lambda i: (i, 0)),
      out_shape=jax.ShapeDtypeStruct((N, C), jnp.float32),
  )(accp, y1, degp, b1, W2)


def _tc3_body(accp_ref, y2_ref, degp_ref, b2_ref, z_ref):
  dinv = _dinv(degp_ref[...])
  z_ref[...] = ((accp_ref[0] + accp_ref[1] + y2_ref[...]) * dinv[:, None]
                + b2_ref[0, :])


def _tc3(accp, y2, degp, b2):
  return pl.pallas_call(
      _tc3_body,
      grid=(N // B,),
      in_specs=[
          pl.BlockSpec((2, B, C), lambda i: (0, i, 0)),
          pl.BlockSpec((B, C), lambda i: (i, 0)),
          pl.BlockSpec((B, 2), lambda i: (i, 0)),
          pl.BlockSpec((1, C), lambda i: (0, 0)),
      ],
      out_specs=pl.BlockSpec((B, C), lambda i: (i, 0)),
      out_shape=jax.ShapeDtypeStruct((N, C), jnp.float32),
  )(accp, y2, degp, b2)


def _tc4_body(za_ref, zb_ref, s_ref):
  s_ref[...] = jnp.sum(za_ref[...] * zb_ref[...], axis=1, keepdims=True)


def _tc4(za, zb):
  BD = 2000
  return pl.pallas_call(
      _tc4_body,
      grid=(EL // BD,),
      in_specs=[
          pl.BlockSpec((BD, C), lambda i: (i, 0)),
          pl.BlockSpec((BD, C), lambda i: (i, 0)),
      ],
      out_specs=pl.BlockSpec((BD, 1), lambda i: (i, 0)),
      out_shape=jax.ShapeDtypeStruct((EL, 1), jnp.float32),
  )(za, zb)


def kernel(x, edge_index, edge_label_index, W1, b1, W2, b2):
  ei = edge_index.astype(jnp.int32)
  el = edge_label_index.astype(jnp.int32)
  src3 = ei[0].reshape(NW, NCHUNK, K)
  dst3 = ei[1].reshape(NW, NCHUNK, K)
  dst_flat = ei[1]
  la2 = el[0].reshape(NCH2, K2)
  lb2 = el[1].reshape(NCH2, K2)
  b1r = b1.reshape(1, C)
  b2r = b2.reshape(1, C)

  degp = _sc_degree(dst_flat).T
  y1 = _tc1(x, W1, degp)
  z0 = jnp.zeros((NP, C), jnp.float32)
  accp1 = _sc_edge(y1, src3, dst3, z0)
  y2 = _tc2(accp1, y1, degp, b1r, W2)
  accp2 = _sc_edge(y2, src3, dst3, z0)
  z = _tc3(accp2, y2, degp, b2r)
  za, zb = _sc_decode_gather(z, la2, lb2)
  scores = _tc4(za, zb).reshape(EL)
  return scores


# speedup vs baseline: 58.4909x; 1.0001x over previous
import functools

import jax
import jax.numpy as jnp
from jax import lax
from jax.experimental import pallas as pl
from jax.experimental.pallas import tpu as pltpu
from jax.experimental.pallas import tpu_sc as plsc

N = 10000
E = 320000
EL = 20000
C = 128
NC, NS, L = 2, 16, 16
NW = NC * NS
EPW = E // NW
K = 125
NCHUNK = EPW // K
SB = 8
RING = 2
NP = 10240
HR = NP // L
RPT = NP // NS
K2 = 80
NCH2 = EL // K2

_mesh = functools.partial(
    plsc.VectorSubcoreMesh, core_axis_name="c", subcore_axis_name="s",
    num_cores=NC, num_subcores=NS)


def _worker_id():
  return lax.axis_index("s") * NC + lax.axis_index("c")


def _deg_body(dst_hbm, degp_hbm, idx_v, hist_v, buf_v, res_v, shared_v):
  cid = lax.axis_index("c")
  sid = lax.axis_index("s")
  wid = _worker_id()
  CW = NP // NS

  @pl.loop(0, NP // L)
  def _zero(i):
    hist_v[pl.ds(i * L, L)] = jnp.zeros((L,), jnp.float32)

  pltpu.sync_copy(dst_hbm.at[pl.ds(wid * EPW, EPW)], idx_v)

  @pl.loop(0, EPW // L, unroll=8)
  def _acc(i):
    idx = idx_v[pl.ds(i * L, L)]
    plsc.addupdate_scatter(hist_v, [idx], jnp.ones((L,), jnp.float32))

  pltpu.sync_copy(hist_v, shared_v.at[sid])
  plsc.subcore_barrier()
  for r in range(NS):
    pltpu.sync_copy(shared_v.at[r, pl.ds(sid * CW, CW)], buf_v.at[r])

  @pl.loop(0, CW // L, unroll=4)
  def _sum(i):
    acc = jnp.zeros((L,), jnp.float32)
    for r in range(NS):
      acc += buf_v[r, pl.ds(i * L, L)]
    res_v[pl.ds(i * L, L)] = acc

  pltpu.sync_copy(res_v, degp_hbm.at[cid, pl.ds(sid * CW, CW)])


def _sc_degree(dst_flat):
  return pl.kernel(
      _deg_body,
      out_type=jax.ShapeDtypeStruct((NC, NP), jnp.float32),
      mesh=_mesh(),
      compiler_params=pltpu.CompilerParams(needs_layout_passes=False),
      scratch_types=[
          pltpu.VMEM((EPW,), jnp.int32),
          pltpu.VMEM((NP,), jnp.float32),
          pltpu.VMEM((NS, NP // NS), jnp.float32),
          pltpu.VMEM((NP // NS,), jnp.float32),
          pltpu.VMEM_SHARED((NS, NP), jnp.float32),
      ],
  )(dst_flat)


def _edge_body(y_hbm, src_hbm, dst_hbm, z0_hbm, accp_hbm,
               src_v, dst_v, rows_v, acc_sh, sg, ss):
  cid = lax.axis_index("c")
  sid = lax.axis_index("s")
  wid = _worker_id()

  pltpu.sync_copy(z0_hbm.at[pl.ds(sid * RPT, RPT)],
                  acc_sh.at[pl.ds(sid * RPT, RPT)])
  plsc.subcore_barrier()

  def stage(ob):
    slot = lax.rem(ob, 2)
    pltpu.sync_copy(src_hbm.at[wid, pl.ds(ob * SB, SB)], src_v.at[slot])
    pltpu.sync_copy(dst_hbm.at[wid, pl.ds(ob * SB, SB)], dst_v.at[slot])

  def g_desc(j, b):
    return pltpu.make_async_copy(
        y_hbm.at[src_v.at[lax.rem(j // SB, 2), lax.rem(j, SB)]],
        rows_v.at[b], sg.at[b])

  def s_start(j, b):
    pltpu.async_copy(
        rows_v.at[b],
        acc_sh.at[dst_v.at[lax.rem(j // SB, 2), lax.rem(j, SB)]],
        ss.at[b], add=True)

  def s_wait(b):
    pltpu.make_async_copy(rows_v.at[b], acc_sh.at[pl.ds(0, K)],
                          ss.at[b]).wait()

  AH = RING // 2

  stage(0)
  for b in range(AH):
    g_desc(b, b).start()

  @pl.loop(0, NCHUNK // RING)
  def _group(q):
    for u in range(RING):
      j = q * RING + u
      g_desc(j, u).wait()
      s_start(j, u)
      jn = j + AH
      bn = (u + AH) % RING

      @pl.when(jn < NCHUNK)
      def _next():
        @pl.when(lax.rem(jn, SB) == 0)
        def _stage():
          stage(jn // SB)

        @pl.when(j >= AH)
        def _free():
          s_wait(bn)

        g_desc(jn, bn).start()

  for u in range(RING):
    s_wait((NCHUNK - RING + u) % RING)

  plsc.subcore_barrier()
  pltpu.sync_copy(acc_sh.at[pl.ds(sid * RPT, RPT)],
                  accp_hbm.at[cid, pl.ds(sid * RPT, RPT)])


def _sc_edge(y, src3, dst3, z0):
  return pl.kernel(
      _edge_body,
      out_type=jax.ShapeDtypeStruct((NC, NP, C), jnp.float32),
      mesh=_mesh(),
      compiler_params=pltpu.CompilerParams(needs_layout_passes=False),
      scratch_types=[
          pltpu.VMEM((2, SB, K), jnp.int32),
          pltpu.VMEM((2, SB, K), jnp.int32),
          pltpu.VMEM((RING, K, C), jnp.float32),
          pltpu.VMEM_SHARED((NP, C), jnp.float32),
          pltpu.SemaphoreType.DMA((RING,)),
          pltpu.SemaphoreType.DMA((RING,)),
      ],
  )(y, src3, dst3, z0)


def _decode_body(z_hbm, la_hbm, lb_hbm, za_hbm, zb_hbm,
                 ia_v, ib_v, ra_v, rb_v, sg):
  wid = _worker_id()
  NI = (NCH2 + NW - 1) // NW

  def stage_and_gather(i, slot):
    c = wid + i * NW

    @pl.when(c < NCH2)
    def _do():
      pltpu.sync_copy(la_hbm.at[c], ia_v.at[slot])
      pltpu.sync_copy(lb_hbm.at[c], ib_v.at[slot])
      pltpu.async_copy(z_hbm.at[ia_v.at[slot]], ra_v.at[slot], sg.at[slot])
      pltpu.async_copy(z_hbm.at[ib_v.at[slot]], rb_v.at[slot], sg.at[slot])

  def wait_rows(slot):
    pltpu.make_async_copy(z_hbm.at[pl.ds(0, K2)], ra_v.at[slot],
                          sg.at[slot]).wait()
    pltpu.make_async_copy(z_hbm.at[pl.ds(0, K2)], rb_v.at[slot],
                          sg.at[slot]).wait()

  stage_and_gather(0, 0)

  @pl.loop(0, NI)
  def _chunk(i):
    c = wid + i * NW
    slot = lax.rem(i, 2)
    stage_and_gather(i + 1, 1 - slot)

    @pl.when(c < NCH2)
    def _do():
      wait_rows(slot)
      pltpu.sync_copy(ra_v.at[slot], za_hbm.at[pl.ds(c * K2, K2)])
      pltpu.sync_copy(rb_v.at[slot], zb_hbm.at[pl.ds(c * K2, K2)])


def _sc_decode_gather(z, la2, lb2):
  return pl.kernel(
      _decode_body,
      out_type=(jax.ShapeDtypeStruct((EL, C), jnp.float32),
                jax.ShapeDtypeStruct((EL, C), jnp.float32)),
      mesh=_mesh(),
      compiler_params=pltpu.CompilerParams(needs_layout_passes=False),
      scratch_types=[
          pltpu.VMEM((2, K2), jnp.int32),
          pltpu.VMEM((2, K2), jnp.int32),
          pltpu.VMEM((2, K2, C), jnp.float32),
          pltpu.VMEM((2, K2, C), jnp.float32),
          pltpu.SemaphoreType.DMA((2,)),
      ],
  )(z, la2, lb2)


B = 1000


def _dinv(degp_blk):
  deg = degp_blk[:, 0] + degp_blk[:, 1] + 1.0
  return lax.rsqrt(deg)


def _tc1_body(x_ref, w_ref, degp_ref, y_ref):
  dinv = _dinv(degp_ref[...])
  xw = jnp.dot(x_ref[...], w_ref[...], preferred_element_type=jnp.float32,
               precision=lax.Precision.HIGHEST)
  y_ref[...] = xw * dinv[:, None]


def _tc1(x, W1, degp):
  return pl.pallas_call(
      _tc1_body,
      grid=(N // B,),
      in_specs=[
          pl.BlockSpec((B, C), lambda i: (i, 0)),
          pl.BlockSpec((C, C), lambda i: (0, 0)),
          pl.BlockSpec((B, 2), lambda i: (i, 0)),
      ],
      out_specs=pl.BlockSpec((B, C), lambda i: (i, 0)),
      out_shape=jax.ShapeDtypeStruct((N, C), jnp.float32),
  )(x, W1, degp)


def _tc2_body(accp_ref, y1_ref, degp_ref, b1_ref, w_ref, y2_ref):
  dinv = _dinv(degp_ref[...])
  pre = (accp_ref[0] + accp_ref[1] + y1_ref[...]) * dinv[:, None] + b1_ref[0, :]
  h = jnp.maximum(pre, 0.0)
  xw = jnp.dot(h, w_ref[...], preferred_element_type=jnp.float32,
               precision=lax.Precision.HIGHEST)
  y2_ref[...] = xw * dinv[:, None]


def _tc2(accp, y1, degp, b1, W2):
  return pl.pallas_call(
      _tc2_body,
      grid=(N // B,),
      in_specs=[
          pl.BlockSpec((2, B, C), lambda i: (0, i, 0)),
          pl.BlockSpec((B, C), lambda i: (i, 0)),
          pl.BlockSpec((B, 2), lambda i: (i, 0)),
          pl.BlockSpec((1, C), lambda i: (0, 0)),
          pl.BlockSpec((C, C), lambda i: (0, 0)),
      ],
      out_specs=pl.BlockSpec((B, C), lambda i: (i, 0)),
      out_shape=jax.ShapeDtypeStruct((N, C), jnp.float32),
  )(accp, y1, degp, b1, W2)


def _tc3_body(accp_ref, y2_ref, degp_ref, b2_ref, z_ref):
  dinv = _dinv(degp_ref[...])
  z_ref[...] = ((accp_ref[0] + accp_ref[1] + y2_ref[...]) * dinv[:, None]
                + b2_ref[0, :])


def _tc3(accp, y2, degp, b2):
  return pl.pallas_call(
      _tc3_body,
      grid=(N // B,),
      in_specs=[
          pl.BlockSpec((2, B, C), lambda i: (0, i, 0)),
          pl.BlockSpec((B, C), lambda i: (i, 0)),
          pl.BlockSpec((B, 2), lambda i: (i, 0)),
          pl.BlockSpec((1, C), lambda i: (0, 0)),
      ],
      out_specs=pl.BlockSpec((B, C), lambda i: (i, 0)),
      out_shape=jax.ShapeDtypeStruct((N, C), jnp.float32),
  )(accp, y2, degp, b2)


def _tc4_body(za_ref, zb_ref, s_ref):
  s_ref[...] = jnp.sum(za_ref[...] * zb_ref[...], axis=1, keepdims=True)


def _tc4(za, zb):
  BD = 2000
  return pl.pallas_call(
      _tc4_body,
      grid=(EL // BD,),
      in_specs=[
          pl.BlockSpec((BD, C), lambda i: (i, 0)),
          pl.BlockSpec((BD, C), lambda i: (i, 0)),
      ],
      out_specs=pl.BlockSpec((BD, 1), lambda i: (i, 0)),
      out_shape=jax.ShapeDtypeStruct((EL, 1), jnp.float32),
  )(za, zb)


def kernel(x, edge_index, edge_label_index, W1, b1, W2, b2):
  ei = edge_index.astype(jnp.int32)
  el = edge_label_index.astype(jnp.int32)
  src3 = ei[0].reshape(NW, NCHUNK, K)
  dst3 = ei[1].reshape(NW, NCHUNK, K)
  dst_flat = ei[1]
  la2 = el[0].reshape(NCH2, K2)
  lb2 = el[1].reshape(NCH2, K2)
  b1r = b1.reshape(1, C)
  b2r = b2.reshape(1, C)

  degp = _sc_degree(dst_flat).T
  y1 = _tc1(x, W1, degp)
  z0 = jnp.zeros((NP, C), jnp.float32)
  accp1 = _sc_edge(y1, src3, dst3, z0)
  y2 = _tc2(accp1, y1, degp, b1r, W2)
  accp2 = _sc_edge(y2, src3, dst3, z0)
  z = _tc3(accp2, y2, degp, b2r)
  za, zb = _sc_decode_gather(z, la2, lb2)
  scores = _tc4(za, zb).reshape(EL)
  return scores
